# baseline (speedup 1.0000x reference)
import jax
import jax.numpy as jnp
from jax import lax
from jax.experimental import pallas as pl
from jax.experimental.pallas import tpu as pltpu
from jax.experimental.pallas import tpu_sc as plsc

NSITES = 64
NTIMES = 8192
NBASE = 2016
NFRAMES = 4096

NC = 2
NS = 16
NW = NC * NS
L = 16

FPW = NFRAMES // NW
FCH = 16
NCH = FPW // FCH
GROUPS = NBASE // L
UNROLL = 6
SUBW = 32
NSUB = NTIMES * 2 * NBASE // 128
B8 = NBASE // 8
FT = NFRAMES // 128


def _tc_clip(gt_ref, out_ref):
    x = gt_ref[...]
    x = (x + jnp.pi) % (2.0 * jnp.pi) - jnp.pi
    out_ref[...] = jnp.clip(x, -jnp.pi, jnp.pi)


def _sc_body(bl_hbm, fr_hbm, cgt_hbm, gi_hbm, gj_hbm,
             idx1_v, idxl_v, cols_v, rows_v, gi_v, gj_v,
             sem_cols, sem_rows0, sem_rows1, sem_out):
    wid = lax.axis_index("s") * NC + lax.axis_index("c")
    fbase = wid * FPW

    pltpu.sync_copy(fr_hbm.at[pl.ds(fbase, FPW)], idx1_v)
    cols_cp = pltpu.async_copy(cgt_hbm.at[idx1_v], cols_v, sem_cols)

    iota = lax.iota(jnp.int32, L)
    iota2 = iota * 2
    io8 = jax.lax.shift_right_logical(iota, 3)
    io7 = jnp.bitwise_and(iota, 7)
    zero_v = jnp.zeros((L,), jnp.int32)

    row_sems = (sem_rows0, sem_rows1)

    def t_of(fw):
        grp = jax.lax.shift_right_logical(fw, 4)
        lane = jnp.bitwise_and(fw, L - 1)
        tv = idx1_v[pl.ds(grp * L, L)]
        return jnp.sum(jnp.where(iota == lane, tv, 0))

    def start_row(fw, buf):
        t = t_of(fw)
        k = jax.lax.shift_right_logical(t * 63, 1)
        idxl_v[buf, pl.ds(0, L)] = k + iota
        idxl_v[buf, pl.ds(L, L)] = k + iota + L
        pltpu.async_copy(
            bl_hbm.at[idxl_v.at[buf]], rows_v.at[buf], row_sems[buf])

    def wait_row(buf):
        pltpu.make_async_copy(
            bl_hbm.at[idxl_v.at[buf]], rows_v.at[buf], row_sems[buf]).wait()

    def compute(fw, buf, fl):
        t = t_of(fw)
        off = jnp.bitwise_and(t, 1) * 64
        buf_vec = jnp.full((L,), buf, jnp.int32)
        fl_vec = jnp.broadcast_to(fl, (L,)).astype(jnp.int32)
        fw_vec = jnp.broadcast_to(fw, (L,)).astype(jnp.int32)

        def body(g6, _):
            for u in range(UNROLL):
                g = g6 * UNROLL + u
                e = off + g * (2 * L) + iota2
                sub = jax.lax.shift_right_logical(e, 7)
                wrd = jnp.bitwise_and(e, 127)
                ivals = plsc.load_gather(rows_v, [buf_vec, sub, wrd])
                jvals = plsc.load_gather(rows_v, [buf_vec, sub, wrd + 1])
                gi = plsc.load_gather(cols_v, [fw_vec, ivals])
                gj = plsc.load_gather(cols_v, [fw_vec, jvals])
                i0 = 2 * g + io8
                plsc.store_scatter(gi_v, [i0, zero_v, io7, fl_vec], gi)
                plsc.store_scatter(gj_v, [i0, zero_v, io7, fl_vec], gj)
            return 0

        lax.fori_loop(0, GROUPS // UNROLL, body, 0)

    start_row(jnp.int32(0), 0)
    cols_cp.wait()
    out_pending = None

    for c in range(NCH):
        if out_pending is not None:
            out_pending[0].wait()
            out_pending[1].wait()

        def pair_body(p, _, c=c):
            fwa = c * FCH + 2 * p
            start_row(fwa + 1, 1)
            wait_row(0)
            compute(fwa, 0, 2 * p)

            @pl.when(fwa + 2 < FPW)
            def _():
                start_row(fwa + 2, 0)

            wait_row(1)
            compute(fwa + 1, 1, 2 * p + 1)
            return 0

        lax.fori_loop(0, FCH // 2, pair_body, 0)

        dst = lambda ref, cc=c: ref.at[
            pl.ds(0, B8), pl.ds(wid, 1), pl.ds(0, 8), pl.ds(cc * FCH, FCH)]
        cp_gi = pltpu.async_copy(gi_v, dst(gi_hbm), sem_out)
        cp_gj = pltpu.async_copy(gj_v, dst(gj_hbm), sem_out)
        out_pending = (cp_gi, cp_gj)

    out_pending[0].wait()
    out_pending[1].wait()


_sc_call = pl.kernel(
    _sc_body,
    out_type=(
        jax.ShapeDtypeStruct((B8, FT, 8, 128), jnp.float32),
        jax.ShapeDtypeStruct((B8, FT, 8, 128), jnp.float32),
    ),
    mesh=plsc.VectorSubcoreMesh(
        core_axis_name="c", subcore_axis_name="s",
        num_cores=NC, num_subcores=NS),
    compiler_params=pltpu.CompilerParams(
        needs_layout_passes=False, use_tc_tiling_on_sc=False),
    scratch_types=[
        pltpu.VMEM((FPW,), jnp.int32),
        pltpu.VMEM((2, SUBW), jnp.int32),
        pltpu.VMEM((FPW, NSITES), jnp.float32),
        pltpu.VMEM((2, SUBW, 128), jnp.int32),
        pltpu.VMEM((B8, 1, 8, FCH), jnp.float32),
        pltpu.VMEM((B8, 1, 8, FCH), jnp.float32),
        pltpu.SemaphoreType.DMA,
        pltpu.SemaphoreType.DMA,
        pltpu.SemaphoreType.DMA,
        pltpu.SemaphoreType.DMA,
    ],
)


def kernel(baselines, frames, gains):
    bl_sub = baselines.reshape(NSUB, 128)
    gt = gains.T
    cgt = pl.pallas_call(
        _tc_clip,
        out_shape=jax.ShapeDtypeStruct((NTIMES, NSITES), jnp.float32),
    )(gt)
    x, y = _sc_call(bl_sub, frames, cgt)
    gi = x.transpose(1, 3, 0, 2).reshape(NFRAMES, NBASE)
    gj = y.transpose(1, 3, 0, 2).reshape(NFRAMES, NBASE)
    return gi, gj


# speedup vs baseline: 29.9291x; 29.9291x over previous
import jax
import jax.numpy as jnp
from jax import lax
from jax.experimental import pallas as pl
from jax.experimental.pallas import tpu as pltpu
from jax.experimental.pallas import tpu_sc as plsc

NSITES = 64
NTIMES = 8192
NBASE = 2016
NFRAMES = 4096

NC = 2
NS = 16
NW = NC * NS
L = 16

FPW = NFRAMES // NW
FCH = 16
NCH = FPW // FCH
GROUPS = NBASE // L
UNROLL = 6
SUBW = 32
NSUB = NTIMES * 2 * NBASE // 128
B8 = NBASE // 8
FT = NFRAMES // 128


def _tc_clip(gt_ref, out_ref):
    x = gt_ref[...]
    x = (x + jnp.pi) % (2.0 * jnp.pi) - jnp.pi
    out_ref[...] = jnp.clip(x, -jnp.pi, jnp.pi)


def _sc_body(bl_hbm, fr_hbm, cgt_hbm, gi_hbm, gj_hbm,
             idx1_v, idxl_v, cols_v, rows_v, gi_v, gj_v,
             sem_cols, sem_rows0, sem_rows1, sem_out):
    wid = lax.axis_index("s") * NC + lax.axis_index("c")
    fbase = wid * FPW

    pltpu.sync_copy(fr_hbm.at[pl.ds(fbase, FPW)], idx1_v)
    cols_cp = pltpu.async_copy(cgt_hbm.at[idx1_v], cols_v, sem_cols)

    iota = lax.iota(jnp.int32, L)
    iota2 = iota * 2
    io8 = jax.lax.shift_right_logical(iota, 3)
    io7 = jnp.bitwise_and(iota, 7)
    zero_v = jnp.zeros((L,), jnp.int32)

    row_sems = (sem_rows0, sem_rows1)

    def t_of(fw):
        grp = jax.lax.shift_right_logical(fw, 4)
        lane = jnp.bitwise_and(fw, L - 1)
        tv = idx1_v[pl.ds(grp * L, L)]
        return jnp.sum(jnp.where(iota == lane, tv, 0))

    def start_row(fw, buf):
        t = t_of(fw)
        idxl_v[buf, pl.ds(0, L)] = jnp.broadcast_to(t, (L,))
        pltpu.async_copy(
            bl_hbm.at[idxl_v.at[buf, pl.ds(0, 1)]], rows_v.at[buf],
            row_sems[buf])

    def wait_row(buf):
        pltpu.make_async_copy(
            bl_hbm.at[idxl_v.at[buf, pl.ds(0, 1)]], rows_v.at[buf],
            row_sems[buf]).wait()

    def compute(fw, buf, fl):
        buf_vec = jnp.full((L,), buf, jnp.int32)
        fl_vec = jnp.broadcast_to(fl, (L,)).astype(jnp.int32)
        fw_vec = jnp.broadcast_to(fw, (L,)).astype(jnp.int32)

        def body(g6, _):
            for u in range(UNROLL):
                g = g6 * UNROLL + u
                e = g * (2 * L) + iota2
                ivals = plsc.load_gather(rows_v, [buf_vec, zero_v, e])
                jvals = plsc.load_gather(rows_v, [buf_vec, zero_v, e + 1])
                gi = plsc.load_gather(cols_v, [fw_vec, ivals])
                gj = plsc.load_gather(cols_v, [fw_vec, jvals])
                i0 = 2 * g + io8
                plsc.store_scatter(gi_v, [i0, zero_v, io7, fl_vec], gi)
                plsc.store_scatter(gj_v, [i0, zero_v, io7, fl_vec], gj)
            return 0

        lax.fori_loop(0, GROUPS // UNROLL, body, 0)

    start_row(jnp.int32(0), 0)
    cols_cp.wait()
    out_pending = None

    for c in range(NCH):
        if out_pending is not None:
            out_pending[0].wait()
            out_pending[1].wait()

        def pair_body(p, _, c=c):
            fwa = c * FCH + 2 * p
            start_row(fwa + 1, 1)
            wait_row(0)
            compute(fwa, 0, 2 * p)

            @pl.when(fwa + 2 < FPW)
            def _():
                start_row(fwa + 2, 0)

            wait_row(1)
            compute(fwa + 1, 1, 2 * p + 1)
            return 0

        lax.fori_loop(0, FCH // 2, pair_body, 0)

        dst = lambda ref, cc=c: ref.at[
            pl.ds(0, B8), pl.ds(wid, 1), pl.ds(0, 8), pl.ds(cc * FCH, FCH)]
        cp_gi = pltpu.async_copy(gi_v, dst(gi_hbm), sem_out)
        cp_gj = pltpu.async_copy(gj_v, dst(gj_hbm), sem_out)
        out_pending = (cp_gi, cp_gj)

    out_pending[0].wait()
    out_pending[1].wait()


_sc_call = pl.kernel(
    _sc_body,
    out_type=(
        jax.ShapeDtypeStruct((B8, FT, 8, 128), jnp.float32),
        jax.ShapeDtypeStruct((B8, FT, 8, 128), jnp.float32),
    ),
    mesh=plsc.VectorSubcoreMesh(
        core_axis_name="c", subcore_axis_name="s",
        num_cores=NC, num_subcores=NS),
    compiler_params=pltpu.CompilerParams(
        needs_layout_passes=False, use_tc_tiling_on_sc=False),
    scratch_types=[
        pltpu.VMEM((FPW,), jnp.int32),
        pltpu.VMEM((2, L), jnp.int32),
        pltpu.VMEM((FPW, NSITES), jnp.float32),
        pltpu.VMEM((2, 1, 2 * NBASE), jnp.int32),
        pltpu.VMEM((B8, 1, 8, FCH), jnp.float32),
        pltpu.VMEM((B8, 1, 8, FCH), jnp.float32),
        pltpu.SemaphoreType.DMA,
        pltpu.SemaphoreType.DMA,
        pltpu.SemaphoreType.DMA,
        pltpu.SemaphoreType.DMA,
    ],
)


def kernel(baselines, frames, gains):
    bl_sub = baselines.reshape(NTIMES, 2 * NBASE)
    gt = gains.T
    cgt = pl.pallas_call(
        _tc_clip,
        out_shape=jax.ShapeDtypeStruct((NTIMES, NSITES), jnp.float32),
    )(gt)
    x, y = _sc_call(bl_sub, frames, cgt)
    gi = x.transpose(1, 3, 0, 2).reshape(NFRAMES, NBASE)
    gj = y.transpose(1, 3, 0, 2).reshape(NFRAMES, NBASE)
    return gi, gj


# speedup vs baseline: 29.9875x; 1.0020x over previous
import jax
import jax.numpy as jnp
from jax import lax
from jax.experimental import pallas as pl
from jax.experimental.pallas import tpu as pltpu
from jax.experimental.pallas import tpu_sc as plsc

NSITES = 64
NTIMES = 8192
NBASE = 2016
NFRAMES = 4096

NC = 2
NS = 16
NW = NC * NS
L = 16

FPW = NFRAMES // NW
FCH = 16
NCH = FPW // FCH
RCH = 4
NRCH = FPW // RCH
GROUPS = NBASE // L
UNROLL = 6
SUBW = 32
NSUB = NTIMES * 2 * NBASE // 128
B8 = NBASE // 8
FT = NFRAMES // 128


def _tc_clip(gt_ref, out_ref):
    x = gt_ref[...]
    x = (x + jnp.pi) % (2.0 * jnp.pi) - jnp.pi
    out_ref[...] = jnp.clip(x, -jnp.pi, jnp.pi)


def _sc_body(bl_hbm, fr_hbm, fr2_hbm, cgt_hbm, gi_hbm, gj_hbm,
             idx1_v, idx2_v, cols_v, rows_v, gi_v, gj_v,
             sem_cols, sem_rows0, sem_rows1, sem_out):
    wid = lax.axis_index("s") * NC + lax.axis_index("c")
    fbase = wid * FPW

    pltpu.sync_copy(fr_hbm.at[pl.ds(fbase, FPW)], idx1_v)
    cols_cp = pltpu.async_copy(cgt_hbm.at[idx1_v], cols_v, sem_cols)
    pltpu.sync_copy(fr2_hbm.at[pl.ds(wid * NRCH, NRCH)], idx2_v)

    iota = lax.iota(jnp.int32, L)
    iota2 = iota * 2
    io8 = jax.lax.shift_right_logical(iota, 3)
    io7 = jnp.bitwise_and(iota, 7)
    zero_v = jnp.zeros((L,), jnp.int32)

    row_sems = (sem_rows0, sem_rows1)

    def start_chunk(rc, buf):
        pltpu.async_copy(
            bl_hbm.at[idx2_v.at[rc]], rows_v.at[buf], row_sems[buf])

    def wait_chunk(buf):
        pltpu.make_async_copy(
            bl_hbm.at[idx2_v.at[0]], rows_v.at[buf], row_sems[buf]).wait()

    def drain_out():
        dst0 = gi_hbm.at[
            pl.ds(0, B8), pl.ds(wid, 1), pl.ds(0, 8), pl.ds(0, FCH)]
        pltpu.make_async_copy(gi_v, dst0, sem_out).wait()
        pltpu.make_async_copy(gj_v, dst0, sem_out).wait()

    def compute(p, buf, fc):
        fw = p * 2 * RCH + buf * RCH + fc
        fl = jnp.bitwise_and(fw, FCH - 1)
        buf_vec = jnp.full((L,), buf, jnp.int32)
        fc_vec = jnp.full((L,), fc, jnp.int32)
        fl_vec = jnp.broadcast_to(fl, (L,)).astype(jnp.int32)
        fw_vec = jnp.broadcast_to(fw, (L,)).astype(jnp.int32)

        def body(g6, _):
            for u in range(UNROLL):
                g = g6 * UNROLL + u
                e = g * (2 * L) + iota2
                ivals = plsc.load_gather(rows_v, [buf_vec, fc_vec, e])
                jvals = plsc.load_gather(rows_v, [buf_vec, fc_vec, e + 1])
                gi = plsc.load_gather(cols_v, [fw_vec, ivals])
                gj = plsc.load_gather(cols_v, [fw_vec, jvals])
                i0 = 2 * g + io8
                plsc.store_scatter(gi_v, [i0, zero_v, io7, fl_vec], gi)
                plsc.store_scatter(gj_v, [i0, zero_v, io7, fl_vec], gj)
            return 0

        lax.fori_loop(0, GROUPS // UNROLL, body, 0)

    start_chunk(jnp.int32(0), 0)
    cols_cp.wait()

    def pair_body(p, _):
        start_chunk(2 * p + 1, 1)
        wait_chunk(0)

        @pl.when(jnp.logical_and(jnp.bitwise_and(p, 1) == 0, p >= 2))
        def _():
            drain_out()

        for fc in range(RCH):
            compute(p, 0, fc)

        @pl.when(2 * p + 2 < NRCH)
        def _():
            start_chunk(2 * p + 2, 0)

        wait_chunk(1)
        for fc in range(RCH):
            compute(p, 1, fc)

        @pl.when(jnp.bitwise_and(p, 1) == 1)
        def _():
            c = jax.lax.shift_right_logical(p, 1)
            dst = lambda ref: ref.at[
                pl.ds(0, B8), pl.ds(wid, 1), pl.ds(0, 8),
                pl.ds(c * FCH, FCH)]
            pltpu.async_copy(gi_v, dst(gi_hbm), sem_out)
            pltpu.async_copy(gj_v, dst(gj_hbm), sem_out)

        return 0

    lax.fori_loop(0, NRCH // 2, pair_body, 0)
    drain_out()


_sc_call = pl.kernel(
    _sc_body,
    out_type=(
        jax.ShapeDtypeStruct((B8, FT, 8, 128), jnp.float32),
        jax.ShapeDtypeStruct((B8, FT, 8, 128), jnp.float32),
    ),
    mesh=plsc.VectorSubcoreMesh(
        core_axis_name="c", subcore_axis_name="s",
        num_cores=NC, num_subcores=NS),
    compiler_params=pltpu.CompilerParams(
        needs_layout_passes=False, use_tc_tiling_on_sc=False),
    scratch_types=[
        pltpu.VMEM((FPW,), jnp.int32),
        pltpu.VMEM((NRCH, RCH), jnp.int32),
        pltpu.VMEM((FPW, NSITES), jnp.float32),
        pltpu.VMEM((2, RCH, 2 * NBASE), jnp.int32),
        pltpu.VMEM((B8, 1, 8, FCH), jnp.float32),
        pltpu.VMEM((B8, 1, 8, FCH), jnp.float32),
        pltpu.SemaphoreType.DMA,
        pltpu.SemaphoreType.DMA,
        pltpu.SemaphoreType.DMA,
        pltpu.SemaphoreType.DMA,
    ],
)


def kernel(baselines, frames, gains):
    bl_flat = baselines.reshape(NTIMES, 2 * NBASE)
    fr2 = frames.reshape(NFRAMES // RCH, RCH)
    gt = gains.T
    cgt = pl.pallas_call(
        _tc_clip,
        out_shape=jax.ShapeDtypeStruct((NTIMES, NSITES), jnp.float32),
    )(gt)
    x, y = _sc_call(bl_flat, frames, fr2, cgt)
    gi = x.transpose(1, 3, 0, 2).reshape(NFRAMES, NBASE)
    gj = y.transpose(1, 3, 0, 2).reshape(NFRAMES, NBASE)
    return gi, gj
